# baseline (speedup 1.0000x reference)
import functools

import jax
import jax.numpy as jnp
from jax import lax
from jax.experimental import pallas as pl
from jax.experimental.pallas import tpu as pltpu
from jax.experimental.pallas import tpu_sc as plsc


@functools.lru_cache(maxsize=None)
def _make_gather(B, S, V, D):
    info = plsc.get_sparse_core_info()
    NC, NS = info.num_cores, info.num_subcores
    NW = NC * NS
    assert B % NW == 0
    BW = B // NW
    mesh = plsc.VectorSubcoreMesh(core_axis_name="c", subcore_axis_name="s")

    @functools.partial(
        pl.kernel,
        mesh=mesh,
        compiler_params=pltpu.CompilerParams(use_tc_tiling_on_sc=False),
        out_type=jax.ShapeDtypeStruct((B, S, D), jnp.float32),
        scratch_types=[
            pltpu.VMEM((2, S), jnp.int32),
            pltpu.VMEM((2, S, D), jnp.float32),
            pltpu.SemaphoreType.DMA((2,)),
            pltpu.SemaphoreType.DMA((2,)),
            pltpu.SemaphoreType.DMA((2,)),
        ],
    )
    def gather_kernel(tok_hbm, table_hbm, out_hbm, idx_v, rows_v,
                      sem_i, sem_g, sem_o):
        wid = lax.axis_index("s") * NC + lax.axis_index("c")
        b0 = wid * BW

        def idx_load(r, slot):
            return pltpu.make_async_copy(
                tok_hbm.at[b0 + r], idx_v.at[slot], sem_i.at[slot])

        def row_gather(slot):
            return pltpu.make_async_copy(
                table_hbm.at[idx_v.at[slot]], rows_v.at[slot], sem_g.at[slot])

        def writeback(r, slot):
            return pltpu.make_async_copy(
                rows_v.at[slot], out_hbm.at[b0 + r], sem_o.at[slot])

        idx_load(0, 0).start()
        idx_load(0, 0).wait()
        row_gather(0).start()
        idx_load(1, 1).start()

        def body(r, carry):
            s_cur = r % 2
            s_prev = 1 - s_cur

            row_gather(s_prev).wait()
            writeback(r - 1, s_prev).start()

            @pl.when(r >= 2)
            def _():
                writeback(r - 2, s_cur).wait()
            idx_load(r, s_cur).wait()
            row_gather(s_cur).start()

            @pl.when(r < BW - 1)
            def _():
                idx_load(r + 1, s_prev).start()
            return carry

        lax.fori_loop(1, BW, body, 0, unroll=False)

        s_last = (BW - 1) % 2
        row_gather(s_last).wait()
        writeback(BW - 1, s_last).start()
        writeback(BW - 2, 1 - s_last).wait()
        writeback(BW - 1, s_last).wait()

    return gather_kernel


def kernel(token_ids, model_idx, table_0):
    B, S = token_ids.shape
    V, D = table_0.shape
    return _make_gather(B, S, V, D)(token_ids, table_0)


# speedup vs baseline: 1.0398x; 1.0398x over previous
import functools

import jax
import jax.numpy as jnp
from jax import lax
from jax.experimental import pallas as pl
from jax.experimental.pallas import tpu as pltpu
from jax.experimental.pallas import tpu_sc as plsc


@functools.lru_cache(maxsize=None)
def _make_gather(B, S, V, D):
    info = plsc.get_sparse_core_info()
    NC, NS = info.num_cores, info.num_subcores
    NW = NC * NS
    assert B % NW == 0
    BW = B // NW
    mesh = plsc.VectorSubcoreMesh(core_axis_name="c", subcore_axis_name="s")

    @functools.partial(
        pl.kernel,
        mesh=mesh,
        compiler_params=pltpu.CompilerParams(use_tc_tiling_on_sc=False),
        out_type=jax.ShapeDtypeStruct((B, S, D), jnp.float32),
        scratch_types=[
            pltpu.VMEM((4, S), jnp.int32),
            pltpu.VMEM((4, S, D), jnp.float32),
            pltpu.SemaphoreType.DMA((4,)),
            pltpu.SemaphoreType.DMA((4,)),
            pltpu.SemaphoreType.DMA((4,)),
        ],
    )
    def gather_kernel(tok_hbm, table_hbm, out_hbm, idx_v, rows_v,
                      sem_i, sem_g, sem_o):
        K = 4
        L = 2
        wid = lax.axis_index("s") * NC + lax.axis_index("c")
        b0 = wid * BW

        def idx_load(r, slot):
            return pltpu.make_async_copy(
                tok_hbm.at[b0 + r], idx_v.at[slot], sem_i.at[slot])

        def row_gather(slot):
            return pltpu.make_async_copy(
                table_hbm.at[idx_v.at[slot]], rows_v.at[slot], sem_g.at[slot])

        def writeback(r, slot):
            return pltpu.make_async_copy(
                rows_v.at[slot], out_hbm.at[b0 + r], sem_o.at[slot])

        for k in range(K):
            idx_load(k, k).start()

        def body(r, carry):
            @pl.when(r < BW)
            def _():
                slot = r % K

                @pl.when(r >= K)
                def _():
                    writeback(r - K, slot).wait()
                idx_load(r, slot).wait()
                row_gather(slot).start()

            @pl.when(r >= L)
            def _():
                g = r - L
                gs = g % K
                row_gather(gs).wait()
                writeback(g, gs).start()

                @pl.when(g + K < BW)
                def _():
                    idx_load(g + K, gs).start()
            return carry

        lax.fori_loop(0, BW + L, body, 0, unroll=False)

        for j in range(BW - K, BW):
            writeback(j, j % K).wait()

    return gather_kernel


def kernel(token_ids, model_idx, table_0):
    B, S = token_ids.shape
    V, D = table_0.shape
    return _make_gather(B, S, V, D)(token_ids, table_0)


# speedup vs baseline: 1.2687x; 1.2201x over previous
import functools

import jax
import jax.numpy as jnp
from jax import lax
from jax.experimental import pallas as pl
from jax.experimental.pallas import tpu as pltpu
from jax.experimental.pallas import tpu_sc as plsc


@functools.lru_cache(maxsize=None)
def _make_gather(N, V, D, CH):
    info = plsc.get_sparse_core_info()
    NC, NS = info.num_cores, info.num_subcores
    NW = NC * NS
    assert N % (NW * CH) == 0
    NB = N // (NW * CH)
    mesh = plsc.VectorSubcoreMesh(core_axis_name="c", subcore_axis_name="s")

    @functools.partial(
        pl.kernel,
        mesh=mesh,
        compiler_params=pltpu.CompilerParams(use_tc_tiling_on_sc=True),
        out_type=jax.ShapeDtypeStruct((N, 128), jnp.float32),
        scratch_types=[
            pltpu.VMEM((4, CH), jnp.int32),
            pltpu.VMEM((4, CH, 128), jnp.float32),
            pltpu.SemaphoreType.DMA((4,)),
            pltpu.SemaphoreType.DMA((4,)),
            pltpu.SemaphoreType.DMA((4,)),
        ],
    )
    def gather_kernel(tok_hbm, table_hbm, out_hbm, idx_v, rows_v,
                      sem_i, sem_g, sem_o):
        K = 4
        L = 2
        wid = lax.axis_index("s") * NC + lax.axis_index("c")
        r0 = wid * (NB * CH)

        def idx_load(j, slot):
            return pltpu.make_async_copy(
                tok_hbm.at[pl.ds(r0 + j * CH, CH)], idx_v.at[slot],
                sem_i.at[slot])

        def row_gather(slot):
            return pltpu.make_async_copy(
                table_hbm.at[idx_v.at[slot]], rows_v.at[slot], sem_g.at[slot])

        def writeback(j, slot):
            return pltpu.make_async_copy(
                rows_v.at[slot], out_hbm.at[pl.ds(r0 + j * CH, CH)],
                sem_o.at[slot])

        for k in range(K):
            idx_load(k, k).start()

        def body(j, carry):
            @pl.when(j < NB)
            def _():
                slot = j % K

                @pl.when(j >= K)
                def _():
                    writeback(j - K, slot).wait()
                idx_load(j, slot).wait()
                row_gather(slot).start()

            @pl.when(j >= L)
            def _():
                g = j - L
                gs = g % K
                row_gather(gs).wait()
                writeback(g, gs).start()

                @pl.when(g + K < NB)
                def _():
                    idx_load(g + K, gs).start()
            return carry

        lax.fori_loop(0, NB + L, body, 0, unroll=False)

        for j in range(NB - K, NB):
            writeback(j, j % K).wait()

    return gather_kernel


def kernel(token_ids, model_idx, table_0):
    B, S = token_ids.shape
    V, D = table_0.shape
    tok_flat = token_ids.reshape(B * S)
    table_p = jnp.pad(table_0, ((0, 0), (0, 128 - D)))
    o = _make_gather(B * S, V, D, 128)(tok_flat, table_p)
    return o.reshape(B, S, 128)[:, :, :D]
